# baseline (speedup 1.0000x reference)
import functools

import jax
import jax.numpy as jnp
from jax import lax
from jax.experimental import pallas as pl
from jax.experimental.pallas import tpu as pltpu
from jax.experimental.pallas import tpu_sc as plsc

N = 10000
E = 320000
NUM_CONT = 16
NUM_CAT = 8
VOCAB = 1000
EMB = 16
HID = 256
F = 128
NC, NS, L = 2, 16, 16
NW = NC * NS
CH = 80
DEGP = 10240

_SC_MESH = plsc.VectorSubcoreMesh(
    core_axis_name="c", subcore_axis_name="s", num_cores=NC, num_subcores=NS)
_SC_PARAMS = pltpu.CompilerParams(use_tc_tiling_on_sc=False)

EPT = (N * NUM_CAT) // NW
ECH = -(-EPT // CH)
EPT_PAD = ECH * CH
EDGES_PT = E // NW
ACH = EDGES_PT // CH
DPT = DEGP // NS


def _emb_body(cat_hbm, offs_hbm, emb_hbm, edge_hbm, out_hbm, deg_hbm,
              degacc, catv, offsv, idxv, outv, dstv, onesv, zerov, sem, dsem):
    c = lax.axis_index("c")
    s = lax.axis_index("s")
    w = c * NS + s
    pltpu.sync_copy(cat_hbm.at[w], catv)
    pltpu.sync_copy(offs_hbm.at[w], offsv)

    def add_body(k, carry):
        i = k // (CH // L)
        jj = k % (CH // L)
        idxv[i, pl.ds(jj * L, L)] = (
            catv[i, pl.ds(jj * L, L)] + offsv[pl.ds(jj * L, L)])
        return carry

    lax.fori_loop(0, ECH * (CH // L), add_body, 0)

    def g_blk(b, carry):
        for u in range(8):
            pltpu.async_copy(
                emb_hbm.at[idxv.at[b * 8 + u]],
                outv.at[pl.ds((b * 8 + u) * CH, CH), :], sem)
        for u in range(8):
            pltpu.make_async_copy(
                emb_hbm.at[idxv.at[b * 8 + u]],
                outv.at[pl.ds((b * 8 + u) * CH, CH), :], sem).wait()
        return carry

    lax.fori_loop(0, ECH // 8, g_blk, 0)
    pltpu.sync_copy(outv.at[pl.ds(0, EPT), :], out_hbm.at[pl.ds(w * EPT, EPT), :])

    def o_body(jj, carry):
        onesv[pl.ds(jj * L, L)] = jnp.ones((L,), jnp.float32)
        zerov[pl.ds(jj * L, L)] = jnp.zeros((L,), jnp.float32)
        return carry

    lax.fori_loop(0, CH // L, o_body, 0)

    def z_body(r, carry):
        pltpu.sync_copy(zerov, degacc.at[pl.ds(s * DPT + r * CH, CH)])
        return carry

    lax.fori_loop(0, DPT // CH, z_body, 0)
    pltpu.sync_copy(edge_hbm.at[1, w], dstv)
    plsc.subcore_barrier()

    def d_body(j, carry):
        pltpu.async_copy(onesv, degacc.at[dstv.at[j]], dsem, add=True)
        return carry

    lax.fori_loop(0, ACH, d_body, 0)

    def d_drain(j, carry):
        pltpu.make_async_copy(onesv, degacc.at[dstv.at[j]], dsem).wait()
        return carry

    lax.fori_loop(0, ACH, d_drain, 0)
    plsc.subcore_barrier()
    pltpu.sync_copy(degacc.at[pl.ds(s * DPT, DPT)],
                    deg_hbm.at[c, pl.ds(s * DPT, DPT)])


_emb_call = pl.kernel(
    _emb_body,
    out_type=(jax.ShapeDtypeStruct((N * NUM_CAT, EMB), jnp.float32),
              jax.ShapeDtypeStruct((NC, DEGP), jnp.float32)),
    mesh=_SC_MESH,
    scratch_types=[
        pltpu.VMEM_SHARED((DEGP,), jnp.float32),
        pltpu.VMEM((ECH, CH), jnp.int32),
        pltpu.VMEM((CH,), jnp.int32),
        pltpu.VMEM((ECH, CH), jnp.int32),
        pltpu.VMEM((EPT_PAD, EMB), jnp.float32),
        pltpu.VMEM((ACH, CH), jnp.int32),
        pltpu.VMEM((CH,), jnp.float32),
        pltpu.VMEM((CH,), jnp.float32),
        pltpu.SemaphoreType.DMA,
        pltpu.SemaphoreType.DMA,
    ],
    compiler_params=_SC_PARAMS,
)

RPT = N // NS
ZFULL = RPT // CH
ZREM = RPT - ZFULL * CH


def _agg_body(h_hbm, edge_hbm, out_hbm, acc, srcv, dstv, r0, r1, r2,
              sem0, sem1, sem2, ssem0, ssem1, ssem2):
    c = lax.axis_index("c")
    s = lax.axis_index("s")
    w = c * NS + s

    def z_body(k, carry):
        i = k // (F // L)
        jj = k % (F // L)
        r0[i, pl.ds(jj * L, L)] = jnp.zeros((L,), jnp.float32)
        return carry

    lax.fori_loop(0, CH * (F // L), z_body, 0)

    def zc_body(r, carry):
        pltpu.sync_copy(r0, acc.at[pl.ds(s * RPT + r * CH, CH), :])
        return carry

    lax.fori_loop(0, ZFULL, zc_body, 0)
    pltpu.sync_copy(r0.at[pl.ds(0, ZREM), :],
                    acc.at[pl.ds(s * RPT + ZFULL * CH, ZREM), :])

    pltpu.sync_copy(edge_hbm.at[0, w], srcv)
    pltpu.sync_copy(edge_hbm.at[1, w], dstv)
    plsc.subcore_barrier()

    bufs = (r0, r1, r2)
    sems = (sem0, sem1, sem2)
    ssems = (ssem0, ssem1, ssem2)

    def gi(j, b):
        return pltpu.async_copy(h_hbm.at[srcv.at[j]], bufs[b], sems[b])

    def gw(j, b):
        pltpu.make_async_copy(h_hbm.at[srcv.at[j]], bufs[b], sems[b]).wait()

    def si(j, b):
        return pltpu.async_copy(bufs[b], acc.at[dstv.at[j]], ssems[b], add=True)

    def sw(j, b):
        pltpu.make_async_copy(bufs[b], acc.at[dstv.at[j]], ssems[b]).wait()

    gi(0, 0)
    gi(1, 1)
    gi(2, 2)

    def e_body(k, carry):
        j0 = 3 * k
        for u in range(3):
            gw(j0 + u, u)
            si(j0 + u, u)
        for u in range(3):
            sw(j0 + u, u)
            gi(j0 + u + 3, u)
        return carry

    KFULL = (ACH - 2) // 3 - 1
    lax.fori_loop(0, KFULL, e_body, 0)
    j0 = 3 * KFULL
    for u in range(3):
        gw(j0 + u, u)
        si(j0 + u, u)
    for u in range(3):
        sw(j0 + u, u)
        if u < 2:
            gi(j0 + u + 3, u)
    for u in range(2):
        gw(j0 + u + 3, u)
        si(j0 + u + 3, u)
    for u in range(2):
        sw(j0 + u + 3, u)

    plsc.subcore_barrier()
    pltpu.sync_copy(acc.at[pl.ds(s * RPT, RPT), :],
                    out_hbm.at[c, pl.ds(s * RPT, RPT), :])


_agg_call = pl.kernel(
    _agg_body,
    out_type=jax.ShapeDtypeStruct((NC, N, F), jnp.float32),
    mesh=_SC_MESH,
    scratch_types=[
        pltpu.VMEM_SHARED((N, F), jnp.float32),
        pltpu.VMEM((ACH, CH), jnp.int32),
        pltpu.VMEM((ACH, CH), jnp.int32),
        pltpu.VMEM((CH, F), jnp.float32),
        pltpu.VMEM((CH, F), jnp.float32),
        pltpu.VMEM((CH, F), jnp.float32),
        pltpu.SemaphoreType.DMA,
        pltpu.SemaphoreType.DMA,
        pltpu.SemaphoreType.DMA,
        pltpu.SemaphoreType.DMA,
        pltpu.SemaphoreType.DMA,
        pltpu.SemaphoreType.DMA,
    ],
    compiler_params=_SC_PARAMS,
)

R = 2000


def _tab_body(x_ref, e_ref, w1x_ref, w1e_ref, b1_ref, w2_ref, b2_ref, o_ref):
    h = jnp.dot(x_ref[...], w1x_ref[...], preferred_element_type=jnp.float32)
    h = h + jnp.dot(e_ref[...], w1e_ref[...], preferred_element_type=jnp.float32)
    h = jnp.maximum(h + b1_ref[...], 0.0)
    o = jnp.dot(h, w2_ref[...], preferred_element_type=jnp.float32) + b2_ref[...]
    o_ref[...] = jnp.maximum(o, 0.0)


_tab_call = pl.pallas_call(
    _tab_body,
    grid=(N // R,),
    in_specs=[
        pl.BlockSpec((R, NUM_CONT), lambda i: (i, 0)),
        pl.BlockSpec((R, F), lambda i: (i, 0)),
        pl.BlockSpec((NUM_CONT, HID), lambda i: (0, 0)),
        pl.BlockSpec((F, HID), lambda i: (0, 0)),
        pl.BlockSpec((1, HID), lambda i: (0, 0)),
        pl.BlockSpec((HID, F), lambda i: (0, 0)),
        pl.BlockSpec((1, F), lambda i: (0, 0)),
    ],
    out_specs=pl.BlockSpec((R, F), lambda i: (i, 0)),
    out_shape=jax.ShapeDtypeStruct((N, F), jnp.float32),
)


def _sage_body(h_ref, p_ref, d_ref, ws_ref, wn_ref, b_ref, o_ref, *, act):
    p = p_ref[...]
    d = d_ref[...]
    rdeg = 1.0 / jnp.maximum(d[0] + d[1], 1.0)
    agg = (p[0] + p[1]) * rdeg
    o = (jnp.dot(h_ref[...], ws_ref[...], preferred_element_type=jnp.float32)
         + jnp.dot(agg, wn_ref[...], preferred_element_type=jnp.float32)
         + b_ref[...])
    if act:
        o = jnp.maximum(o, 0.0)
    o_ref[...] = o


def _make_sage(act):
    return pl.pallas_call(
        functools.partial(_sage_body, act=act),
        grid=(N // R,),
        in_specs=[
            pl.BlockSpec((R, F), lambda i: (i, 0)),
            pl.BlockSpec((NC, R, F), lambda i: (0, i, 0)),
            pl.BlockSpec((NC, R, 1), lambda i: (0, i, 0)),
            pl.BlockSpec((F, F), lambda i: (0, 0)),
            pl.BlockSpec((F, F), lambda i: (0, 0)),
            pl.BlockSpec((1, F), lambda i: (0, 0)),
        ],
        out_specs=pl.BlockSpec((R, F), lambda i: (i, 0)),
        out_shape=jax.ShapeDtypeStruct((N, F), jnp.float32),
    )


_sage1_call = _make_sage(act=True)
_sage2_call = _make_sage(act=False)


def kernel(x, cat, edge_index, emb, W1, b1, W2, b2, Ws1, Wn1, sb1, Ws2, Wn2, sb2):
    cat32 = cat.astype(jnp.int32).reshape(NW, EPT)
    cat_pad = jnp.pad(cat32, ((0, 0), (0, EPT_PAD - EPT))).reshape(NW, ECH, CH)
    offs = ((jnp.arange(NW, dtype=jnp.int32)[:, None] * EPT
             + jnp.arange(CH, dtype=jnp.int32)[None, :]) % NUM_CAT) * VOCAB
    emb_flat = emb.reshape(NUM_CAT * VOCAB, EMB)
    edge_r = edge_index.astype(jnp.int32).reshape(2, NW, ACH, CH)

    e_flat, degp = _emb_call(cat_pad, offs, emb_flat, edge_r)
    e = e_flat.reshape(N, NUM_CAT * EMB)
    dp = degp[:, :N].reshape(NC, N, 1)

    h0 = _tab_call(x, e, W1[:NUM_CONT], W1[NUM_CONT:],
                   b1.reshape(1, HID), W2, b2.reshape(1, F))

    p = _agg_call(h0, edge_r)
    h1 = _sage1_call(h0, p, dp, Ws1, Wn1, sb1.reshape(1, F))
    q = _agg_call(h1, edge_r)
    out = _sage2_call(h1, q, dp, Ws2, Wn2, sb2.reshape(1, F))
    return out


# speedup vs baseline: 1.1874x; 1.1874x over previous
import functools

import jax
import jax.numpy as jnp
from jax import lax
from jax.experimental import pallas as pl
from jax.experimental.pallas import tpu as pltpu
from jax.experimental.pallas import tpu_sc as plsc

N = 10000
E = 320000
NUM_CONT = 16
NUM_CAT = 8
VOCAB = 1000
EMB = 16
HID = 256
F = 128
NC, NS, L = 2, 16, 16
NW = NC * NS
CH = 80
DEGP = 10240

_SC_MESH = plsc.VectorSubcoreMesh(
    core_axis_name="c", subcore_axis_name="s", num_cores=NC, num_subcores=NS)
_SC_PARAMS = pltpu.CompilerParams(use_tc_tiling_on_sc=False)

EPT = (N * NUM_CAT) // NW
ECH = -(-EPT // CH)
EPT_PAD = ECH * CH
EDGES_PT = E // NW
ACH = EDGES_PT // CH
DPT = DEGP // NS


def _emb_body(cat_hbm, offs_hbm, emb_hbm, edge_hbm, out_hbm, deg_hbm,
              degacc, catv, offsv, idxv, outv, dstv, onesv, zerov, sem, dsem):
    c = lax.axis_index("c")
    s = lax.axis_index("s")
    w = c * NS + s
    pltpu.sync_copy(cat_hbm.at[w], catv)
    pltpu.sync_copy(offs_hbm.at[w], offsv)

    def add_body(k, carry):
        i = k // (CH // L)
        jj = k % (CH // L)
        idxv[i, pl.ds(jj * L, L)] = (
            catv[i, pl.ds(jj * L, L)] + offsv[pl.ds(jj * L, L)])
        return carry

    lax.fori_loop(0, ECH * (CH // L), add_body, 0)

    def g_blk(b, carry):
        for u in range(8):
            pltpu.async_copy(
                emb_hbm.at[idxv.at[b * 8 + u]],
                outv.at[pl.ds((b * 8 + u) * CH, CH), :], sem)
        for u in range(8):
            pltpu.make_async_copy(
                emb_hbm.at[idxv.at[b * 8 + u]],
                outv.at[pl.ds((b * 8 + u) * CH, CH), :], sem).wait()
        return carry

    lax.fori_loop(0, ECH // 8, g_blk, 0)
    pltpu.sync_copy(outv.at[pl.ds(0, EPT), :], out_hbm.at[pl.ds(w * EPT, EPT), :])

    def o_body(jj, carry):
        onesv[pl.ds(jj * L, L)] = jnp.ones((L,), jnp.float32)
        zerov[pl.ds(jj * L, L)] = jnp.zeros((L,), jnp.float32)
        return carry

    lax.fori_loop(0, CH // L, o_body, 0)

    def z_body(r, carry):
        pltpu.sync_copy(zerov, degacc.at[pl.ds(s * DPT + r * CH, CH)])
        return carry

    lax.fori_loop(0, DPT // CH, z_body, 0)
    pltpu.sync_copy(edge_hbm.at[1, w], dstv)
    plsc.subcore_barrier()

    def d_body(j, carry):
        pltpu.async_copy(onesv, degacc.at[dstv.at[j]], dsem, add=True)
        return carry

    lax.fori_loop(0, ACH, d_body, 0)

    def d_drain(j, carry):
        pltpu.make_async_copy(onesv, degacc.at[dstv.at[j]], dsem).wait()
        return carry

    lax.fori_loop(0, ACH, d_drain, 0)
    plsc.subcore_barrier()
    pltpu.sync_copy(degacc.at[pl.ds(s * DPT, DPT)],
                    deg_hbm.at[c, pl.ds(s * DPT, DPT)])


_emb_call = pl.kernel(
    _emb_body,
    out_type=(jax.ShapeDtypeStruct((N * NUM_CAT, EMB), jnp.float32),
              jax.ShapeDtypeStruct((NC, DEGP), jnp.float32)),
    mesh=_SC_MESH,
    scratch_types=[
        pltpu.VMEM_SHARED((DEGP,), jnp.float32),
        pltpu.VMEM((ECH, CH), jnp.int32),
        pltpu.VMEM((CH,), jnp.int32),
        pltpu.VMEM((ECH, CH), jnp.int32),
        pltpu.VMEM((EPT_PAD, EMB), jnp.float32),
        pltpu.VMEM((ACH, CH), jnp.int32),
        pltpu.VMEM((CH,), jnp.float32),
        pltpu.VMEM((CH,), jnp.float32),
        pltpu.SemaphoreType.DMA,
        pltpu.SemaphoreType.DMA,
    ],
    compiler_params=_SC_PARAMS,
)

RPT = N // NS
ZFULL = RPT // CH
ZREM = RPT - ZFULL * CH


def _agg_body(h_hbm, edge_hbm, out_hbm, acc, srcv, dstv, r0, r1, r2,
              sem0, sem1, sem2):
    c = lax.axis_index("c")
    s = lax.axis_index("s")
    w = c * NS + s

    def z_body(k, carry):
        i = k // (F // L)
        jj = k % (F // L)
        r0[i, pl.ds(jj * L, L)] = jnp.zeros((L,), jnp.float32)
        return carry

    lax.fori_loop(0, CH * (F // L), z_body, 0)

    def zc_body(r, carry):
        pltpu.sync_copy(r0, acc.at[pl.ds(s * RPT + r * CH, CH), :])
        return carry

    lax.fori_loop(0, ZFULL, zc_body, 0)
    pltpu.sync_copy(r0.at[pl.ds(0, ZREM), :],
                    acc.at[pl.ds(s * RPT + ZFULL * CH, ZREM), :])

    pltpu.sync_copy(edge_hbm.at[0, w], srcv)
    pltpu.sync_copy(edge_hbm.at[1, w], dstv)
    plsc.subcore_barrier()

    bufs = (r0, r1, r2)
    sems = (sem0, sem1, sem2)

    def gi(j, b):
        return pltpu.async_copy(h_hbm.at[srcv.at[j]], bufs[b], sems[b])

    def gw(j, b):
        pltpu.make_async_copy(h_hbm.at[srcv.at[j]], bufs[b], sems[b]).wait()

    def sc(j, b):
        pltpu.sync_copy(bufs[b], acc.at[dstv.at[j]], add=True)

    gi(0, 0)
    gi(1, 1)

    def e_body(k, carry):
        j0 = 3 * k
        gi(j0 + 2, 2)
        gw(j0, 0)
        sc(j0, 0)
        gi(j0 + 3, 0)
        gw(j0 + 1, 1)
        sc(j0 + 1, 1)
        gi(j0 + 4, 1)
        gw(j0 + 2, 2)
        sc(j0 + 2, 2)
        return carry

    lax.fori_loop(0, (ACH - 2) // 3, e_body, 0)
    gw(ACH - 2, 0)
    sc(ACH - 2, 0)
    gw(ACH - 1, 1)
    sc(ACH - 1, 1)

    plsc.subcore_barrier()
    pltpu.sync_copy(acc.at[pl.ds(s * RPT, RPT), :],
                    out_hbm.at[c, pl.ds(s * RPT, RPT), :])


_agg_call = pl.kernel(
    _agg_body,
    out_type=jax.ShapeDtypeStruct((NC, N, F), jnp.float32),
    mesh=_SC_MESH,
    scratch_types=[
        pltpu.VMEM_SHARED((N, F), jnp.float32),
        pltpu.VMEM((ACH, CH), jnp.int32),
        pltpu.VMEM((ACH, CH), jnp.int32),
        pltpu.VMEM((CH, F), jnp.float32),
        pltpu.VMEM((CH, F), jnp.float32),
        pltpu.VMEM((CH, F), jnp.float32),
        pltpu.SemaphoreType.DMA,
        pltpu.SemaphoreType.DMA,
        pltpu.SemaphoreType.DMA,
    ],
    compiler_params=_SC_PARAMS,
)

R = 2000


def _tab_body(x_ref, e_ref, w1x_ref, w1e_ref, b1_ref, w2_ref, b2_ref, o_ref):
    h = jnp.dot(x_ref[...], w1x_ref[...], preferred_element_type=jnp.float32)
    h = h + jnp.dot(e_ref[...], w1e_ref[...], preferred_element_type=jnp.float32)
    h = jnp.maximum(h + b1_ref[...], 0.0)
    o = jnp.dot(h, w2_ref[...], preferred_element_type=jnp.float32) + b2_ref[...]
    o_ref[...] = jnp.maximum(o, 0.0)


_tab_call = pl.pallas_call(
    _tab_body,
    grid=(N // R,),
    in_specs=[
        pl.BlockSpec((R, NUM_CONT), lambda i: (i, 0)),
        pl.BlockSpec((R, F), lambda i: (i, 0)),
        pl.BlockSpec((NUM_CONT, HID), lambda i: (0, 0)),
        pl.BlockSpec((F, HID), lambda i: (0, 0)),
        pl.BlockSpec((1, HID), lambda i: (0, 0)),
        pl.BlockSpec((HID, F), lambda i: (0, 0)),
        pl.BlockSpec((1, F), lambda i: (0, 0)),
    ],
    out_specs=pl.BlockSpec((R, F), lambda i: (i, 0)),
    out_shape=jax.ShapeDtypeStruct((N, F), jnp.float32),
)


def _sage_body(h_ref, p_ref, d_ref, ws_ref, wn_ref, b_ref, o_ref, *, act):
    p = p_ref[...]
    d = d_ref[...]
    rdeg = 1.0 / jnp.maximum(d[0] + d[1], 1.0)
    agg = (p[0] + p[1]) * rdeg
    o = (jnp.dot(h_ref[...], ws_ref[...], preferred_element_type=jnp.float32)
         + jnp.dot(agg, wn_ref[...], preferred_element_type=jnp.float32)
         + b_ref[...])
    if act:
        o = jnp.maximum(o, 0.0)
    o_ref[...] = o


def _make_sage(act):
    return pl.pallas_call(
        functools.partial(_sage_body, act=act),
        grid=(N // R,),
        in_specs=[
            pl.BlockSpec((R, F), lambda i: (i, 0)),
            pl.BlockSpec((NC, R, F), lambda i: (0, i, 0)),
            pl.BlockSpec((NC, R, 1), lambda i: (0, i, 0)),
            pl.BlockSpec((F, F), lambda i: (0, 0)),
            pl.BlockSpec((F, F), lambda i: (0, 0)),
            pl.BlockSpec((1, F), lambda i: (0, 0)),
        ],
        out_specs=pl.BlockSpec((R, F), lambda i: (i, 0)),
        out_shape=jax.ShapeDtypeStruct((N, F), jnp.float32),
    )


_sage1_call = _make_sage(act=True)
_sage2_call = _make_sage(act=False)


def kernel(x, cat, edge_index, emb, W1, b1, W2, b2, Ws1, Wn1, sb1, Ws2, Wn2, sb2):
    cat32 = cat.astype(jnp.int32).reshape(NW, EPT)
    cat_pad = jnp.pad(cat32, ((0, 0), (0, EPT_PAD - EPT))).reshape(NW, ECH, CH)
    offs = ((jnp.arange(NW, dtype=jnp.int32)[:, None] * EPT
             + jnp.arange(CH, dtype=jnp.int32)[None, :]) % NUM_CAT) * VOCAB
    emb_flat = emb.reshape(NUM_CAT * VOCAB, EMB)
    edge_r = edge_index.astype(jnp.int32).reshape(2, NW, ACH, CH)

    e_flat, degp = _emb_call(cat_pad, offs, emb_flat, edge_r)
    e = e_flat.reshape(N, NUM_CAT * EMB)
    dp = degp[:, :N].reshape(NC, N, 1)

    h0 = _tab_call(x, e, W1[:NUM_CONT], W1[NUM_CONT:],
                   b1.reshape(1, HID), W2, b2.reshape(1, F))

    p = _agg_call(h0, edge_r)
    h1 = _sage1_call(h0, p, dp, Ws1, Wn1, sb1.reshape(1, F))
    q = _agg_call(h1, edge_r)
    out = _sage2_call(h1, q, dp, Ws2, Wn2, sb2.reshape(1, F))
    return out


# speedup vs baseline: 1.2252x; 1.0318x over previous
import functools

import jax
import jax.numpy as jnp
from jax import lax
from jax.experimental import pallas as pl
from jax.experimental.pallas import tpu as pltpu
from jax.experimental.pallas import tpu_sc as plsc

N = 10000
E = 320000
NUM_CONT = 16
NUM_CAT = 8
VOCAB = 1000
EMB = 16
HID = 256
F = 128
NC, NS, L = 2, 16, 16
NW = NC * NS
CH = 80
DEGP = 10240

_SC_MESH = plsc.VectorSubcoreMesh(
    core_axis_name="c", subcore_axis_name="s", num_cores=NC, num_subcores=NS)
_SC_PARAMS = pltpu.CompilerParams(use_tc_tiling_on_sc=False)

EPT = (N * NUM_CAT) // NW
ECH = -(-EPT // CH)
EPT_PAD = ECH * CH
EDGES_PT = E // NW
ACH = EDGES_PT // CH
DPT = DEGP // NS


DSC = (ACH - 1) // (ECH // 8)


def _emb_body(cat_hbm, offs_hbm, emb_hbm, edge_hbm, out_hbm, deg_hbm,
              degacc, catv, offsv, idxv, outv, dstv, onesv, zerov, sem, dsem):
    c = lax.axis_index("c")
    s = lax.axis_index("s")
    w = c * NS + s
    pltpu.async_copy(edge_hbm.at[1, w], dstv, dsem)

    def o_body(jj, carry):
        onesv[pl.ds(jj * L, L)] = jnp.ones((L,), jnp.float32)
        zerov[pl.ds(jj * L, L)] = jnp.zeros((L,), jnp.float32)
        return carry

    lax.fori_loop(0, CH // L, o_body, 0)

    def z_body(r, carry):
        pltpu.sync_copy(zerov, degacc.at[pl.ds(s * DPT + r * CH, CH)])
        return carry

    lax.fori_loop(0, DPT // CH, z_body, 0)

    pltpu.sync_copy(cat_hbm.at[w], catv)
    pltpu.sync_copy(offs_hbm.at[w], offsv)

    def add_body(k, carry):
        i = k // (CH // L)
        jj = k % (CH // L)
        idxv[i, pl.ds(jj * L, L)] = (
            catv[i, pl.ds(jj * L, L)] + offsv[pl.ds(jj * L, L)])
        return carry

    lax.fori_loop(0, ECH * (CH // L), add_body, 0)
    pltpu.make_async_copy(edge_hbm.at[1, w], dstv, dsem).wait()
    plsc.subcore_barrier()

    def g_blk(b, carry):
        for u in range(8):
            pltpu.async_copy(
                emb_hbm.at[idxv.at[b * 8 + u]],
                outv.at[pl.ds((b * 8 + u) * CH, CH), :], sem)
        for u in range(DSC):
            pltpu.async_copy(onesv, degacc.at[dstv.at[b * DSC + u]], dsem,
                             add=True)
        for u in range(8):
            pltpu.make_async_copy(
                emb_hbm.at[idxv.at[b * 8 + u]],
                outv.at[pl.ds((b * 8 + u) * CH, CH), :], sem).wait()
        return carry

    lax.fori_loop(0, ECH // 8, g_blk, 0)
    for j in range((ECH // 8) * DSC, ACH):
        pltpu.async_copy(onesv, degacc.at[dstv.at[j]], dsem, add=True)
    pltpu.sync_copy(outv.at[pl.ds(0, EPT), :], out_hbm.at[pl.ds(w * EPT, EPT), :])

    def d_drain(j, carry):
        pltpu.make_async_copy(onesv, degacc.at[dstv.at[j]], dsem).wait()
        return carry

    lax.fori_loop(0, ACH, d_drain, 0)
    plsc.subcore_barrier()
    pltpu.sync_copy(degacc.at[pl.ds(s * DPT, DPT)],
                    deg_hbm.at[c, pl.ds(s * DPT, DPT)])


_emb_call = pl.kernel(
    _emb_body,
    out_type=(jax.ShapeDtypeStruct((N * NUM_CAT, EMB), jnp.float32),
              jax.ShapeDtypeStruct((NC, DEGP), jnp.float32)),
    mesh=_SC_MESH,
    scratch_types=[
        pltpu.VMEM_SHARED((DEGP,), jnp.float32),
        pltpu.VMEM((ECH, CH), jnp.int32),
        pltpu.VMEM((CH,), jnp.int32),
        pltpu.VMEM((ECH, CH), jnp.int32),
        pltpu.VMEM((EPT_PAD, EMB), jnp.float32),
        pltpu.VMEM((ACH, CH), jnp.int32),
        pltpu.VMEM((CH,), jnp.float32),
        pltpu.VMEM((CH,), jnp.float32),
        pltpu.SemaphoreType.DMA,
        pltpu.SemaphoreType.DMA,
    ],
    compiler_params=_SC_PARAMS,
)

RPT = N // NS
ZFULL = RPT // CH
ZREM = RPT - ZFULL * CH


def _agg_body(h_hbm, edge_hbm, out_hbm, acc, srcv, dstv, r0, r1, r2,
              sem0, sem1, sem2):
    c = lax.axis_index("c")
    s = lax.axis_index("s")
    w = c * NS + s

    def z_body(k, carry):
        i = k // (F // L)
        jj = k % (F // L)
        r0[i, pl.ds(jj * L, L)] = jnp.zeros((L,), jnp.float32)
        return carry

    lax.fori_loop(0, CH * (F // L), z_body, 0)

    def zc_body(r, carry):
        pltpu.async_copy(r0, acc.at[pl.ds(s * RPT + r * CH, CH), :], sem1)
        return carry

    lax.fori_loop(0, ZFULL, zc_body, 0)
    pltpu.async_copy(r0.at[pl.ds(0, ZREM), :],
                     acc.at[pl.ds(s * RPT + ZFULL * CH, ZREM), :], sem1)
    pltpu.async_copy(edge_hbm.at[0, w], srcv, sem2)
    pltpu.async_copy(edge_hbm.at[1, w], dstv, sem2)
    pltpu.make_async_copy(edge_hbm.at[0, w], srcv, sem2).wait()
    pltpu.make_async_copy(edge_hbm.at[1, w], dstv, sem2).wait()

    def zd_body(r, carry):
        pltpu.make_async_copy(r0, acc.at[pl.ds(s * RPT + r * CH, CH), :],
                              sem1).wait()
        return carry

    lax.fori_loop(0, ZFULL, zd_body, 0)
    pltpu.make_async_copy(r0.at[pl.ds(0, ZREM), :],
                          acc.at[pl.ds(s * RPT + ZFULL * CH, ZREM), :],
                          sem1).wait()

    bufs = (r0, r1, r2)
    sems = (sem0, sem1, sem2)

    def gi(j, b):
        return pltpu.async_copy(h_hbm.at[srcv.at[j]], bufs[b], sems[b])

    def gw(j, b):
        pltpu.make_async_copy(h_hbm.at[srcv.at[j]], bufs[b], sems[b]).wait()

    def sc(j, b):
        pltpu.sync_copy(bufs[b], acc.at[dstv.at[j]], add=True)

    gi(0, 0)
    gi(1, 1)
    plsc.subcore_barrier()

    def e_body(k, carry):
        j0 = 3 * k
        gi(j0 + 2, 2)
        gw(j0, 0)
        sc(j0, 0)
        gi(j0 + 3, 0)
        gw(j0 + 1, 1)
        sc(j0 + 1, 1)
        gi(j0 + 4, 1)
        gw(j0 + 2, 2)
        sc(j0 + 2, 2)
        return carry

    lax.fori_loop(0, (ACH - 2) // 3, e_body, 0)
    gw(ACH - 2, 0)
    sc(ACH - 2, 0)
    gw(ACH - 1, 1)
    sc(ACH - 1, 1)

    plsc.subcore_barrier()
    pltpu.sync_copy(acc.at[pl.ds(s * RPT, RPT), :],
                    out_hbm.at[c, pl.ds(s * RPT, RPT), :])


_agg_call = pl.kernel(
    _agg_body,
    out_type=jax.ShapeDtypeStruct((NC, N, F), jnp.float32),
    mesh=_SC_MESH,
    scratch_types=[
        pltpu.VMEM_SHARED((N, F), jnp.float32),
        pltpu.VMEM((ACH, CH), jnp.int32),
        pltpu.VMEM((ACH, CH), jnp.int32),
        pltpu.VMEM((CH, F), jnp.float32),
        pltpu.VMEM((CH, F), jnp.float32),
        pltpu.VMEM((CH, F), jnp.float32),
        pltpu.SemaphoreType.DMA,
        pltpu.SemaphoreType.DMA,
        pltpu.SemaphoreType.DMA,
    ],
    compiler_params=_SC_PARAMS,
)

R = 2000


def _tab_body(x_ref, e_ref, w1x_ref, w1e_ref, b1_ref, w2_ref, b2_ref, o_ref):
    h = jnp.dot(x_ref[...], w1x_ref[...], preferred_element_type=jnp.float32)
    h = h + jnp.dot(e_ref[...], w1e_ref[...], preferred_element_type=jnp.float32)
    h = jnp.maximum(h + b1_ref[...], 0.0)
    o = jnp.dot(h, w2_ref[...], preferred_element_type=jnp.float32) + b2_ref[...]
    o_ref[...] = jnp.maximum(o, 0.0)


_tab_call = pl.pallas_call(
    _tab_body,
    grid=(N // R,),
    in_specs=[
        pl.BlockSpec((R, NUM_CONT), lambda i: (i, 0)),
        pl.BlockSpec((R, F), lambda i: (i, 0)),
        pl.BlockSpec((NUM_CONT, HID), lambda i: (0, 0)),
        pl.BlockSpec((F, HID), lambda i: (0, 0)),
        pl.BlockSpec((1, HID), lambda i: (0, 0)),
        pl.BlockSpec((HID, F), lambda i: (0, 0)),
        pl.BlockSpec((1, F), lambda i: (0, 0)),
    ],
    out_specs=pl.BlockSpec((R, F), lambda i: (i, 0)),
    out_shape=jax.ShapeDtypeStruct((N, F), jnp.float32),
)


def _sage_body(h_ref, p_ref, d_ref, ws_ref, wn_ref, b_ref, o_ref, *, act):
    p = p_ref[...]
    d = d_ref[...]
    rdeg = 1.0 / jnp.maximum(d[0] + d[1], 1.0)
    agg = (p[0] + p[1]) * rdeg
    o = (jnp.dot(h_ref[...], ws_ref[...], preferred_element_type=jnp.float32)
         + jnp.dot(agg, wn_ref[...], preferred_element_type=jnp.float32)
         + b_ref[...])
    if act:
        o = jnp.maximum(o, 0.0)
    o_ref[...] = o


def _make_sage(act):
    return pl.pallas_call(
        functools.partial(_sage_body, act=act),
        grid=(N // R,),
        in_specs=[
            pl.BlockSpec((R, F), lambda i: (i, 0)),
            pl.BlockSpec((NC, R, F), lambda i: (0, i, 0)),
            pl.BlockSpec((NC, R, 1), lambda i: (0, i, 0)),
            pl.BlockSpec((F, F), lambda i: (0, 0)),
            pl.BlockSpec((F, F), lambda i: (0, 0)),
            pl.BlockSpec((1, F), lambda i: (0, 0)),
        ],
        out_specs=pl.BlockSpec((R, F), lambda i: (i, 0)),
        out_shape=jax.ShapeDtypeStruct((N, F), jnp.float32),
    )


_sage1_call = _make_sage(act=True)
_sage2_call = _make_sage(act=False)


def kernel(x, cat, edge_index, emb, W1, b1, W2, b2, Ws1, Wn1, sb1, Ws2, Wn2, sb2):
    cat32 = cat.astype(jnp.int32).reshape(NW, EPT)
    cat_pad = jnp.pad(cat32, ((0, 0), (0, EPT_PAD - EPT))).reshape(NW, ECH, CH)
    offs = ((jnp.arange(NW, dtype=jnp.int32)[:, None] * EPT
             + jnp.arange(CH, dtype=jnp.int32)[None, :]) % NUM_CAT) * VOCAB
    emb_flat = emb.reshape(NUM_CAT * VOCAB, EMB)
    edge_r = edge_index.astype(jnp.int32).reshape(2, NW, ACH, CH)

    e_flat, degp = _emb_call(cat_pad, offs, emb_flat, edge_r)
    e = e_flat.reshape(N, NUM_CAT * EMB)
    dp = degp[:, :N].reshape(NC, N, 1)

    h0 = _tab_call(x, e, W1[:NUM_CONT], W1[NUM_CONT:],
                   b1.reshape(1, HID), W2, b2.reshape(1, F))

    p = _agg_call(h0, edge_r)
    h1 = _sage1_call(h0, p, dp, Ws1, Wn1, sb1.reshape(1, F))
    q = _agg_call(h1, edge_r)
    out = _sage2_call(h1, q, dp, Ws2, Wn2, sb2.reshape(1, F))
    return out
